# baseline (speedup 1.0000x reference)
import functools

import jax
import jax.numpy as jnp
from jax import lax
from jax.experimental import pallas as pl
from jax.experimental.pallas import tpu as pltpu
from jax.experimental.pallas import tpu_sc as plsc


def _make_sc_gather(B, N, K, NC, NS, L):
    NW = NC * NS
    CHN = 256
    NPAD = (N + CHN - 1) // CHN * CHN
    NCHT = NPAD // CHN
    NCHF = N // CHN
    assert K == L and N % 8 == 0
    TRIPS = (NCHT + NW - 1) // NW

    mesh = plsc.VectorSubcoreMesh(core_axis_name="c", subcore_axis_name="s")

    @functools.partial(
        pl.kernel,
        out_type=(jax.ShapeDtypeStruct((B, K, NPAD), jnp.int32),
                  jax.ShapeDtypeStruct((B, K, NPAD), jnp.float32)),
        mesh=mesh,
        compiler_params=pltpu.CompilerParams(needs_layout_passes=False),
        scratch_types=[
            pltpu.VMEM((NPAD,), jnp.int32),
            pltpu.VMEM((NPAD,), jnp.float32),
            pltpu.VMEM((2, K, CHN), jnp.int32),
            pltpu.VMEM((2, K, CHN), jnp.int32),
            pltpu.VMEM((2, K, CHN), jnp.float32),
            pltpu.SemaphoreType.DMA,
            pltpu.SemaphoreType.DMA,
        ],
    )
    def k(xyp_hbm, z_hbm, nbr_hbm, nbrtail_hbm, gxy_hbm, gz_hbm,
          tblxy, tblz, idx2, gxy2, gz2, sin, sout):
        wid = lax.axis_index("s") * NC + lax.axis_index("c")

        def issue_idx(b, ch, par):
            @pl.when(ch < NCHF)
            def _():
                pltpu.async_copy(nbr_hbm.at[b, :, pl.ds(ch * CHN, CHN)],
                                 idx2.at[par], sin)

            @pl.when(ch == NCHF)
            def _():
                pltpu.async_copy(nbrtail_hbm.at[b], idx2.at[par], sin)

        def batch_body(b, _):
            pltpu.sync_copy(xyp_hbm.at[pl.ds(b * N, N)], tblxy.at[pl.ds(0, N)])
            pltpu.sync_copy(z_hbm.at[pl.ds(b * N, N)], tblz.at[pl.ds(0, N)])
            issue_idx(b, wid, 0)

            def pair_body(i2, _):
                for par in (0, 1):
                    r = i2 * 2 + par
                    ch = wid + r * NW

                    @pl.when(ch < NCHT)
                    def _(r=r, ch=ch, par=par):
                        pltpu.make_async_copy(
                            nbr_hbm.at[b, :, pl.ds(0, CHN)], idx2.at[par], sin
                        ).wait()
                        issue_idx(b, ch + NW, par ^ 1)

                        @pl.when(r >= 2)
                        def _():
                            pltpu.make_async_copy(
                                gxy_hbm.at[b, :, pl.ds(0, CHN)],
                                gxy2.at[par], sout).wait()
                            pltpu.make_async_copy(
                                gz_hbm.at[b, :, pl.ds(0, CHN)],
                                gz2.at[par], sout).wait()

                        @plsc.parallel_loop(0, CHN // L, unroll=8)
                        def jj_body(jj):
                            for kk in range(K):
                                idxv = idx2[par, kk, pl.ds(jj * L, L)]
                                gxy2[par, kk, pl.ds(jj * L, L)] = (
                                    plsc.load_gather(tblxy, [idxv]))
                                gz2[par, kk, pl.ds(jj * L, L)] = (
                                    plsc.load_gather(tblz, [idxv]))

                        n0 = ch * CHN
                        pltpu.async_copy(gxy2.at[par],
                                         gxy_hbm.at[b, :, pl.ds(n0, CHN)], sout)
                        pltpu.async_copy(gz2.at[par],
                                         gz_hbm.at[b, :, pl.ds(n0, CHN)], sout)

                return 0

            lax.fori_loop(0, (TRIPS + 1) // 2, pair_body, 0)

            tw = (NCHT - wid + NW - 1) // NW
            for thresh in (1, 2):
                @pl.when(tw >= thresh)
                def _():
                    pltpu.make_async_copy(
                        gxy_hbm.at[b, :, pl.ds(0, CHN)], gxy2.at[0], sout).wait()
                    pltpu.make_async_copy(
                        gz_hbm.at[b, :, pl.ds(0, CHN)], gz2.at[0], sout).wait()

            return 0

        lax.fori_loop(0, B, batch_body, 0)

    return k, NPAD, NCHF * CHN


def _unpack(pxy_ref, gz_ref):
    pxy = pxy_ref[0]
    return (
        lax.bitcast_convert_type(pxy & jnp.int32(-65536), jnp.float32),
        lax.bitcast_convert_type(pxy << 16, jnp.float32),
        gz_ref[0],
    )


def _make_tc_stage(B, N, K, which):
    BN = 25088
    NB = (N + BN - 1) // BN
    K_ = K

    def own_body(acc, xyz3_ref, out_ref):
        for c in range(3):
            out_ref[0, c] = jnp.broadcast_to(xyz3_ref[0, c][None, :], (K_, BN))

    def dist_body(acc, dist_ref, out_ref):
        out_ref[0, 0] = dist_ref[0]

    def nb_body(acc, gxy_ref, gz_ref, out_ref):
        nbs = _unpack(gxy_ref, gz_ref)
        for c in range(3):
            out_ref[0, c] = nbs[c]

    def diff_body(acc, xyz3_ref, gxy_ref, gz_ref, out_ref):
        nbs = _unpack(gxy_ref, gz_ref)
        for c in range(3):
            bc = jnp.broadcast_to(xyz3_ref[0, c][None, :], (K_, BN))
            out_ref[0, c] = bc - nbs[c]

    xyz3_spec = pl.BlockSpec((1, 3, BN), lambda b, i: (b, 0, i))
    kn_spec = pl.BlockSpec((1, K, BN), lambda b, i: (b, 0, i))
    body, ins, cblk, cidx = {
        "own": (own_body, [xyz3_spec], 3, 0),
        "dist": (dist_body, [kn_spec], 1, 9),
        "nb": (nb_body, [kn_spec, kn_spec], 3, 1),
        "diff": (diff_body, [xyz3_spec, kn_spec, kn_spec], 3, 2),
    }[which]

    first = which == "own"
    if first:
        def body2(*refs):
            return body(None, *refs)
    else:
        body2 = body

    return pl.pallas_call(
        body2,
        grid=(B, NB),
        in_specs=([] if first else [pl.BlockSpec(memory_space=pl.ANY)]) + ins,
        out_specs=pl.BlockSpec((1, cblk, K, BN), lambda b, i, c=cidx: (b, c, 0, i)),
        out_shape=jax.ShapeDtypeStruct((B, 10, K, N), jnp.float32),
        input_output_aliases={} if first else {0: 0},
    )


def kernel(xyz, neighbors, distances):
    B, N, K = neighbors.shape
    info = plsc.get_sparse_core_info()
    sc_gather, NPAD, T0 = _make_sc_gather(
        B, N, K, info.num_cores, info.num_subcores, info.num_lanes)

    xi = lax.bitcast_convert_type(xyz[:, :, 0], jnp.uint32)
    yi = lax.bitcast_convert_type(xyz[:, :, 1], jnp.uint32)
    xyp = lax.bitcast_convert_type(
        ((xi + 0x8000) & jnp.uint32(0xFFFF0000)) | ((yi + 0x8000) >> 16),
        jnp.int32).reshape(B * N)
    zflat = xyz[:, :, 2].reshape(B * N)

    xyz3 = jnp.transpose(xyz, (0, 2, 1))
    nbr_t = jnp.transpose(neighbors.astype(jnp.int32), (0, 2, 1))
    nbr_tail = jnp.pad(nbr_t[:, :, T0:], ((0, 0), (0, 0), (0, NPAD - N)))
    dist_t = jnp.transpose(distances, (0, 2, 1))

    gxy, gz = sc_gather(xyp, zflat, nbr_t, nbr_tail)
    acc = _make_tc_stage(B, N, K, "own")(xyz3)
    acc = _make_tc_stage(B, N, K, "dist")(acc, dist_t)
    acc = _make_tc_stage(B, N, K, "nb")(acc, gxy, gz)
    acc = _make_tc_stage(B, N, K, "diff")(acc, xyz3, gxy, gz)
    return jnp.transpose(acc, (0, 1, 3, 2))


# speedup vs baseline: 1.0776x; 1.0776x over previous
import functools

import jax
import jax.numpy as jnp
from jax import lax
from jax.experimental import pallas as pl
from jax.experimental.pallas import tpu as pltpu
from jax.experimental.pallas import tpu_sc as plsc


def _make_sc_gather(B, N, K, NC, NS, L):
    NW = NC * NS
    CHN = 256
    NPAD = (N + CHN - 1) // CHN * CHN
    NCHT = NPAD // CHN
    NCHF = N // CHN
    assert K == L and N % 8 == 0
    TRIPS = (NCHT + NW - 1) // NW

    mesh = plsc.VectorSubcoreMesh(core_axis_name="c", subcore_axis_name="s")

    @functools.partial(
        pl.kernel,
        out_type=(jax.ShapeDtypeStruct((B, K, NPAD), jnp.int32),
                  jax.ShapeDtypeStruct((B, K, NPAD), jnp.float32)),
        mesh=mesh,
        compiler_params=pltpu.CompilerParams(needs_layout_passes=False),
        scratch_types=[
            pltpu.VMEM((NPAD,), jnp.int32),
            pltpu.VMEM((NPAD,), jnp.float32),
            pltpu.VMEM((2, K, CHN), jnp.int32),
            pltpu.VMEM((2, K, CHN), jnp.int32),
            pltpu.VMEM((2, K, CHN), jnp.float32),
            pltpu.SemaphoreType.DMA,
            pltpu.SemaphoreType.DMA,
        ],
    )
    def k(xyp_hbm, z_hbm, nbr_hbm, nbrtail_hbm, gxy_hbm, gz_hbm,
          tblxy, tblz, idx2, gxy2, gz2, sin, sout):
        wid = lax.axis_index("s") * NC + lax.axis_index("c")

        def issue_idx(b, ch, par):
            @pl.when(ch < NCHF)
            def _():
                pltpu.async_copy(nbr_hbm.at[b, :, pl.ds(ch * CHN, CHN)],
                                 idx2.at[par], sin)

            @pl.when(ch == NCHF)
            def _():
                pltpu.async_copy(nbrtail_hbm.at[b], idx2.at[par], sin)

        def batch_body(b, _):
            pltpu.sync_copy(xyp_hbm.at[pl.ds(b * N, N)], tblxy.at[pl.ds(0, N)])
            pltpu.sync_copy(z_hbm.at[pl.ds(b * N, N)], tblz.at[pl.ds(0, N)])
            issue_idx(b, wid, 0)

            def pair_body(i2, _):
                for par in (0, 1):
                    r = i2 * 2 + par
                    ch = wid + r * NW

                    @pl.when(ch < NCHT)
                    def _(r=r, ch=ch, par=par):
                        pltpu.make_async_copy(
                            nbr_hbm.at[b, :, pl.ds(0, CHN)], idx2.at[par], sin
                        ).wait()
                        issue_idx(b, ch + NW, par ^ 1)

                        @pl.when(r >= 2)
                        def _():
                            pltpu.make_async_copy(
                                gxy_hbm.at[b, :, pl.ds(0, CHN)],
                                gxy2.at[par], sout).wait()
                            pltpu.make_async_copy(
                                gz_hbm.at[b, :, pl.ds(0, CHN)],
                                gz2.at[par], sout).wait()

                        @plsc.parallel_loop(0, CHN // L, unroll=8)
                        def jj_body(jj):
                            for kk in range(K):
                                idxv = idx2[par, kk, pl.ds(jj * L, L)]
                                gxy2[par, kk, pl.ds(jj * L, L)] = (
                                    plsc.load_gather(tblxy, [idxv]))
                                gz2[par, kk, pl.ds(jj * L, L)] = (
                                    plsc.load_gather(tblz, [idxv]))

                        n0 = ch * CHN
                        pltpu.async_copy(gxy2.at[par],
                                         gxy_hbm.at[b, :, pl.ds(n0, CHN)], sout)
                        pltpu.async_copy(gz2.at[par],
                                         gz_hbm.at[b, :, pl.ds(n0, CHN)], sout)

                return 0

            lax.fori_loop(0, (TRIPS + 1) // 2, pair_body, 0)

            tw = (NCHT - wid + NW - 1) // NW
            for thresh in (1, 2):
                @pl.when(tw >= thresh)
                def _():
                    pltpu.make_async_copy(
                        gxy_hbm.at[b, :, pl.ds(0, CHN)], gxy2.at[0], sout).wait()
                    pltpu.make_async_copy(
                        gz_hbm.at[b, :, pl.ds(0, CHN)], gz2.at[0], sout).wait()

            return 0

        lax.fori_loop(0, B, batch_body, 0)

    return k, NPAD, NCHF * CHN


def _unpack(pxy_ref, gz_ref):
    pxy = pxy_ref[0]
    return (
        lax.bitcast_convert_type(pxy & jnp.int32(-65536), jnp.float32),
        lax.bitcast_convert_type(pxy << 16, jnp.float32),
        gz_ref[0],
    )


def _make_tc_stage(B, N, K, which):
    BN = 25088
    NB = (N + BN - 1) // BN
    K_ = K

    def own_body(acc, xyz3_ref, out_ref):
        for c in range(3):
            out_ref[0, c] = jnp.broadcast_to(xyz3_ref[0, c][None, :], (K_, BN))

    def dist_body(acc, dist_ref, out_ref):
        out_ref[0, 0] = dist_ref[0]

    def main9_body(acc, xyz3_ref, gxy_ref, gz_ref, out_ref):
        nbs = _unpack(gxy_ref, gz_ref)
        for c in range(3):
            bc = jnp.broadcast_to(xyz3_ref[0, c][None, :], (K_, BN))
            out_ref[0, c] = bc
            out_ref[0, 3 + c] = nbs[c]
            out_ref[0, 6 + c] = bc - nbs[c]

    def nb_body(acc, gxy_ref, gz_ref, out_ref):
        nbs = _unpack(gxy_ref, gz_ref)
        for c in range(3):
            out_ref[0, c] = nbs[c]

    def diff_body(acc, xyz3_ref, gxy_ref, gz_ref, out_ref):
        nbs = _unpack(gxy_ref, gz_ref)
        for c in range(3):
            bc = jnp.broadcast_to(xyz3_ref[0, c][None, :], (K_, BN))
            out_ref[0, c] = bc - nbs[c]

    xyz3_spec = pl.BlockSpec((1, 3, BN), lambda b, i: (b, 0, i))
    kn_spec = pl.BlockSpec((1, K, BN), lambda b, i: (b, 0, i))
    body, ins, cblk, cidx = {
        "own": (own_body, [xyz3_spec], 3, 0),
        "dist": (dist_body, [kn_spec], 1, 9),
        "nb": (nb_body, [kn_spec, kn_spec], 3, 1),
        "diff": (diff_body, [xyz3_spec, kn_spec, kn_spec], 3, 2),
        "main9": (main9_body, [xyz3_spec, kn_spec, kn_spec], 9, 0),
    }[which]

    first = which == "dist"
    if first:
        def body2(*refs):
            return body(None, *refs)
    else:
        body2 = body

    return pl.pallas_call(
        body2,
        grid=(B, NB),
        in_specs=([] if first else [pl.BlockSpec(memory_space=pl.ANY)]) + ins,
        out_specs=pl.BlockSpec((1, cblk, K, BN), lambda b, i, c=cidx: (b, c, 0, i)),
        out_shape=jax.ShapeDtypeStruct((B, 10, K, N), jnp.float32),
        input_output_aliases={} if first else {0: 0},
    )


def kernel(xyz, neighbors, distances):
    B, N, K = neighbors.shape
    info = plsc.get_sparse_core_info()
    sc_gather, NPAD, T0 = _make_sc_gather(
        B, N, K, info.num_cores, info.num_subcores, info.num_lanes)

    xi = lax.bitcast_convert_type(xyz[:, :, 0], jnp.uint32)
    yi = lax.bitcast_convert_type(xyz[:, :, 1], jnp.uint32)
    xyp = lax.bitcast_convert_type(
        ((xi + 0x8000) & jnp.uint32(0xFFFF0000)) | ((yi + 0x8000) >> 16),
        jnp.int32).reshape(B * N)
    zflat = xyz[:, :, 2].reshape(B * N)

    xyz3 = jnp.transpose(xyz, (0, 2, 1))
    nbr_t = jnp.transpose(neighbors.astype(jnp.int32), (0, 2, 1))
    nbr_tail = jnp.pad(nbr_t[:, :, T0:], ((0, 0), (0, 0), (0, NPAD - N)))
    dist_t = jnp.transpose(distances, (0, 2, 1))

    gxy, gz = sc_gather(xyp, zflat, nbr_t, nbr_tail)
    acc = _make_tc_stage(B, N, K, "dist")(dist_t)
    acc = _make_tc_stage(B, N, K, "main9")(acc, xyz3, gxy, gz)
    return jnp.transpose(acc, (0, 1, 3, 2))
